# baseline (speedup 1.0000x reference)
import functools

import jax
import jax.numpy as jnp
from jax import lax
from jax.experimental import pallas as pl
import jax.experimental.pallas.tpu as pltpu
from jax.experimental.pallas import tpu_sc as plsc

N_EMBED = 1024
E = 8
EP = 128
TOP_K = 2
B_ = 4
S_ = 8192
S_SC = 2048
S_TC = S_ - S_SC
BS = 2048
NS_TC = S_TC // BS

NW = 32
WPB = NW // B_
R_W = S_SC // WPB
CH = 4
NPAIR = R_W // (2 * CH)


def _tc_stream_kernel(x_ref, wn_ref, bn_ref, acc_out_ref, acc_ref):
    b = pl.program_id(0)
    s = pl.program_id(1)

    @pl.when((b == 0) & (s == 0))
    def _init():
        acc_ref[...] = jnp.zeros_like(acc_ref)

    x = x_ref[0]
    yt = lax.dot_general(wn_ref[...], x,
                         (((1,), (1,)), ((), ())),
                         preferred_element_type=jnp.float32)
    yt = yt + bn_ref[...]
    sp = jnp.maximum(yt, 0.0) + jnp.log1p(jnp.exp(-jnp.abs(yt)))
    part = jnp.sum(sp, axis=1, keepdims=True)
    lane = lax.broadcasted_iota(jnp.int32, (E, EP), 1)
    acc_ref[...] += jnp.where(lane == b, part, 0.0)

    @pl.when((b == B_ - 1) & (s == NS_TC - 1))
    def _flush():
        acc_out_ref[...] = acc_ref[...]


def _tc_partial(mh_output, W_noise, b_noise):
    return pl.pallas_call(
        _tc_stream_kernel,
        grid=(B_, NS_TC),
        in_specs=[
            pl.BlockSpec((1, BS, N_EMBED), lambda b, s: (b, s, 0)),
            pl.BlockSpec((E, N_EMBED), lambda b, s: (0, 0)),
            pl.BlockSpec((E, 1), lambda b, s: (0, 0)),
        ],
        out_specs=pl.BlockSpec((E, EP), lambda b, s: (0, 0)),
        out_shape=jax.ShapeDtypeStruct((E, EP), jnp.float32),
        scratch_shapes=[pltpu.VMEM((E, EP), jnp.float32)],
    )(mh_output, W_noise, b_noise[:, None])


_GDN = lax.GatherDimensionNumbers(
    offset_dims=(), collapsed_slice_dims=(0,), start_index_map=(0,))


def _perm(v, idx):
    return lax.gather(v, idx[:, None], _GDN, (1,),
                      mode=lax.GatherScatterMode.PROMISE_IN_BOUNDS)


def _sc_body(x_hbm, wn_hbm, y_hbm, xa, xb, wv, yv, sema, semb):
    wid = lax.axis_index("c") * 16 + lax.axis_index("s")
    b = wid // WPB
    k = wid % WPB
    row0 = S_TC + k * R_W

    pltpu.sync_copy(wn_hbm, wv)
    lanes = lax.iota(jnp.int32, 16)
    sh8 = (lanes + 8) % 16
    sh4 = (lanes + 4) % 16
    sh2 = (lanes + 2) % 16
    sh1 = (lanes + 1) % 16

    def start(buf, sem, chunk):
        pltpu.make_async_copy(
            x_hbm.at[b, pl.ds(row0 + chunk * CH, CH), :], buf, sem).start()

    def wait(buf, sem, chunk):
        pltpu.make_async_copy(
            x_hbm.at[b, pl.ds(row0 + chunk * CH, CH), :], buf, sem).wait()

    def compute(buf, chunk):
        UNROLL = 4

        def cbody(ci, accs):
            accs = list(accs)
            for u in range(UNROLL):
                c = ci * UNROLL + u
                ws = [wv[e, pl.ds(c * 16, 16)] for e in range(E)]
                for r in range(CH):
                    xv = buf[r, pl.ds(c * 16, 16)]
                    for e in range(E):
                        accs[r * E + e] = accs[r * E + e] + xv * ws[e]
            return tuple(accs)

        init = tuple(jnp.zeros((16,), jnp.float32) for _ in range(CH * E))
        accs = lax.fori_loop(0, N_EMBED // (16 * UNROLL), cbody, init)
        out0 = jnp.zeros((16,), jnp.float32)
        out1 = jnp.zeros((16,), jnp.float32)
        for r in range(CH):
            for e in range(E):
                j = r * E + e
                a = accs[j]
                a = a + _perm(a, sh8)
                a = a + _perm(a, sh4)
                a = a + _perm(a, sh2)
                a = a + _perm(a, sh1)
                if j < 16:
                    out0 = jnp.where(lanes == j, a, out0)
                else:
                    out1 = jnp.where(lanes == (j - 16), a, out1)
        yv[pl.ds(chunk * (CH * E), 16)] = out0
        yv[pl.ds(chunk * (CH * E) + 16, 16)] = out1

    start(xa, sema, 0)
    start(xb, semb, 1)

    def pair(i, carry):
        wait(xa, sema, 2 * i)
        compute(xa, 2 * i)

        @pl.when(i < NPAIR - 1)
        def _():
            start(xa, sema, 2 * i + 2)

        wait(xb, semb, 2 * i + 1)
        compute(xb, 2 * i + 1)

        @pl.when(i < NPAIR - 1)
        def _():
            start(xb, semb, 2 * i + 3)

        return carry

    lax.fori_loop(0, NPAIR, pair, 0)
    pltpu.sync_copy(yv, y_hbm.at[b, pl.ds(k * R_W * E, R_W * E)])


def _sc_partial(mh_output, W_noise):
    mesh = plsc.VectorSubcoreMesh(core_axis_name="c", subcore_axis_name="s")
    kern = functools.partial(
        pl.kernel,
        mesh=mesh,
        out_type=jax.ShapeDtypeStruct((B_, WPB * R_W * E), jnp.float32),
        scratch_types=[
            pltpu.VMEM((CH, N_EMBED), jnp.float32),
            pltpu.VMEM((CH, N_EMBED), jnp.float32),
            pltpu.VMEM((E, N_EMBED), jnp.float32),
            pltpu.VMEM((R_W * E,), jnp.float32),
            pltpu.SemaphoreType.DMA,
            pltpu.SemaphoreType.DMA,
        ],
    )(_sc_body)
    return kern(mh_output, W_noise)


_YROW = S_SC * E // EP


def _combine_kernel(ysc_ref, acc_ref, avg_ref, wr_ref, br_ref, bn_ref,
                    ns_ref, router_ref, idx_ref):
    acc = acc_ref[...]
    bn_tile = jnp.concatenate([bn_ref[...]] * (EP // E), axis=1)
    sc_rows = []
    for b in range(B_):
        z = ysc_ref[b] + bn_tile
        sp = jnp.maximum(z, 0.0) + jnp.log1p(jnp.exp(-jnp.abs(z)))
        colsum = jnp.sum(sp, axis=0, keepdims=True)
        tot = colsum[:, 0:E]
        for g in range(1, EP // E):
            tot = tot + colsum[:, g * E:(g + 1) * E]
        sc_rows.append(tot)
    msc = jnp.concatenate(sc_rows, axis=0)

    mean = (jnp.transpose(acc)[:B_, :E] + msc) * (1.0 / S_)
    logits = lax.dot_general(avg_ref[...], wr_ref[...],
                             (((1,), (1,)), ((), ())),
                             preferred_element_type=jnp.float32)
    noisy = logits + br_ref[...] + ns_ref[...] * mean
    col = lax.broadcasted_iota(jnp.int32, (B_, E), 1)
    neg = jnp.float32(-1e30)
    m1 = jnp.max(noisy, axis=1, keepdims=True)
    i1 = jnp.min(jnp.where(noisy == m1, col, E), axis=1, keepdims=True)
    rest = jnp.where(col == i1, neg, noisy)
    m2 = jnp.max(rest, axis=1, keepdims=True)
    i2 = jnp.min(jnp.where(rest == m2, col, E), axis=1, keepdims=True)
    d = jnp.exp(m2 - m1)
    p1 = 1.0 / (1.0 + d)
    p2 = d / (1.0 + d)
    router_ref[...] = jnp.where(col == i1, p1,
                                jnp.where(col == i2, p2, 0.0))
    idx_ref[:, 0:1] = i1
    idx_ref[:, 1:2] = i2


def _combine(y_sc, acc_tc, mh_output_avg, W_route, b_route, b_noise,
             noise_sample):
    y3 = y_sc.reshape(B_, _YROW, EP)
    return pl.pallas_call(
        _combine_kernel,
        in_specs=[
            pl.BlockSpec((B_, _YROW, EP), lambda: (0, 0, 0)),
            pl.BlockSpec((E, EP), lambda: (0, 0)),
            pl.BlockSpec((B_, N_EMBED), lambda: (0, 0)),
            pl.BlockSpec((E, N_EMBED), lambda: (0, 0)),
            pl.BlockSpec((1, E), lambda: (0, 0)),
            pl.BlockSpec((1, E), lambda: (0, 0)),
            pl.BlockSpec((B_, E), lambda: (0, 0)),
        ],
        out_specs=[
            pl.BlockSpec((B_, E), lambda: (0, 0)),
            pl.BlockSpec((B_, TOP_K), lambda: (0, 0)),
        ],
        out_shape=[
            jax.ShapeDtypeStruct((B_, E), jnp.float32),
            jax.ShapeDtypeStruct((B_, TOP_K), jnp.int32),
        ],
    )(y3, acc_tc, mh_output_avg, W_route, b_route[None, :],
      b_noise[None, :], noise_sample)


def kernel(mh_output, mh_output_avg, W_route, b_route, W_noise, b_noise):
    noise_sample = jax.random.normal(jax.random.key(42), (B_, E),
                                     dtype=jnp.float32)
    y_sc = _sc_partial(mh_output, W_noise)
    acc_tc = _tc_partial(mh_output, W_noise, b_noise)
    return _combine(y_sc, acc_tc, mh_output_avg, W_route, b_route, b_noise,
                    noise_sample)


# speedup vs baseline: 3.4383x; 3.4383x over previous
import jax
import jax.numpy as jnp
from jax.experimental import pallas as pl
import jax.experimental.pallas.tpu as pltpu

N_EMBED = 1024
E = 8
EP = 128
TOP_K = 2
B_ = 4
S_ = 8192
BS = 1024
NS = S_ // BS

def _router_kernel(x_ref, avg_ref, wr_ref, br_ref, wn_ref, bn_ref, ns_ref,
                   router_ref, idx_ref, acc_ref):
    b = pl.program_id(0)
    s = pl.program_id(1)

    @pl.when((b == 0) & (s == 0))
    def _init():
        acc_ref[...] = jnp.zeros_like(acc_ref)

    x = x_ref[0]
    yt = jax.lax.dot_general(wn_ref[...], x,
                             (((1,), (1,)), ((), ())),
                             preferred_element_type=jnp.float32)
    yt = yt + bn_ref[...]
    sp = jnp.maximum(yt, 0.0) + jnp.log1p(jnp.exp(-jnp.abs(yt)))
    part = jnp.sum(sp, axis=1, keepdims=True)
    lane = jax.lax.broadcasted_iota(jnp.int32, (E, EP), 1)
    acc_ref[...] += jnp.where(lane == b, part, 0.0)

    @pl.when((b == B_ - 1) & (s == NS - 1))
    def _epilogue():
        mean = jnp.transpose(acc_ref[...])[:B_, :E] * (1.0 / S_)
        logits = jax.lax.dot_general(avg_ref[...], wr_ref[...],
                                     (((1,), (1,)), ((), ())),
                                     preferred_element_type=jnp.float32)
        noisy = logits + br_ref[...] + ns_ref[...] * mean
        col = jax.lax.broadcasted_iota(jnp.int32, (B_, E), 1)
        neg = jnp.float32(-1e30)
        m1 = jnp.max(noisy, axis=1, keepdims=True)
        i1 = jnp.min(jnp.where(noisy == m1, col, E), axis=1, keepdims=True)
        rest = jnp.where(col == i1, neg, noisy)
        m2 = jnp.max(rest, axis=1, keepdims=True)
        i2 = jnp.min(jnp.where(rest == m2, col, E), axis=1, keepdims=True)
        d = jnp.exp(m2 - m1)
        p1 = 1.0 / (1.0 + d)
        p2 = d / (1.0 + d)
        router_ref[...] = jnp.where(col == i1, p1,
                                    jnp.where(col == i2, p2, 0.0))
        idx_ref[:, 0:1] = i1
        idx_ref[:, 1:2] = i2


def kernel(mh_output, mh_output_avg, W_route, b_route, W_noise, b_noise):
    noise_sample = jax.random.normal(jax.random.key(42), (B_, E),
                                     dtype=jnp.float32)
    return pl.pallas_call(
        _router_kernel,
        grid=(B_, NS),
        in_specs=[
            pl.BlockSpec((1, BS, N_EMBED), lambda b, s: (b, s, 0)),
            pl.BlockSpec((B_, N_EMBED), lambda b, s: (0, 0)),
            pl.BlockSpec((E, N_EMBED), lambda b, s: (0, 0)),
            pl.BlockSpec((1, E), lambda b, s: (0, 0)),
            pl.BlockSpec((E, N_EMBED), lambda b, s: (0, 0)),
            pl.BlockSpec((E, 1), lambda b, s: (0, 0)),
            pl.BlockSpec((B_, E), lambda b, s: (0, 0)),
        ],
        out_specs=[
            pl.BlockSpec((B_, E), lambda b, s: (0, 0)),
            pl.BlockSpec((B_, TOP_K), lambda b, s: (0, 0)),
        ],
        out_shape=[
            jax.ShapeDtypeStruct((B_, E), jnp.float32),
            jax.ShapeDtypeStruct((B_, TOP_K), jnp.int32),
        ],
        scratch_shapes=[pltpu.VMEM((E, EP), jnp.float32)],
    )(mh_output, mh_output_avg, W_route, b_route[None, :], W_noise,
      b_noise[:, None], noise_sample)


# speedup vs baseline: 4.1023x; 1.1931x over previous
import jax
import jax.numpy as jnp
from jax.experimental import pallas as pl
import jax.experimental.pallas.tpu as pltpu

N_EMBED = 1024
E = 8
EP = 128
TOP_K = 2
B_ = 4
S_ = 8192
BS = 2048
NS = S_ // BS

def _router_kernel(x_ref, avg_ref, wr_ref, br_ref, wn_ref, bn_ref, ns_ref,
                   router_ref, idx_ref, acc_ref):
    b = pl.program_id(0)
    s = pl.program_id(1)

    @pl.when((b == 0) & (s == 0))
    def _init():
        acc_ref[...] = jnp.zeros_like(acc_ref)

    x = x_ref[0]
    yt = jax.lax.dot_general(wn_ref[...], x,
                             (((1,), (1,)), ((), ())),
                             preferred_element_type=jnp.float32)
    yt = yt + bn_ref[...]
    sp = jnp.maximum(yt, 0.0) + jnp.log1p(jnp.exp(-jnp.abs(yt)))
    part = jnp.sum(sp, axis=1, keepdims=True)
    lane = jax.lax.broadcasted_iota(jnp.int32, (E, EP), 1)
    acc_ref[...] += jnp.where(lane == b, part, 0.0)

    @pl.when((b == B_ - 1) & (s == NS - 1))
    def _epilogue():
        mean = jnp.transpose(acc_ref[...])[:B_, :E] * (1.0 / S_)
        logits = jax.lax.dot_general(avg_ref[...], wr_ref[...],
                                     (((1,), (1,)), ((), ())),
                                     preferred_element_type=jnp.float32)
        noisy = logits + br_ref[...] + ns_ref[...] * mean
        col = jax.lax.broadcasted_iota(jnp.int32, (B_, E), 1)
        neg = jnp.float32(-1e30)
        m1 = jnp.max(noisy, axis=1, keepdims=True)
        i1 = jnp.min(jnp.where(noisy == m1, col, E), axis=1, keepdims=True)
        rest = jnp.where(col == i1, neg, noisy)
        m2 = jnp.max(rest, axis=1, keepdims=True)
        i2 = jnp.min(jnp.where(rest == m2, col, E), axis=1, keepdims=True)
        d = jnp.exp(m2 - m1)
        p1 = 1.0 / (1.0 + d)
        p2 = d / (1.0 + d)
        router_ref[...] = jnp.where(col == i1, p1,
                                    jnp.where(col == i2, p2, 0.0))
        idx_ref[:, 0:1] = i1
        idx_ref[:, 1:2] = i2


def kernel(mh_output, mh_output_avg, W_route, b_route, W_noise, b_noise):
    noise_sample = jax.random.normal(jax.random.key(42), (B_, E),
                                     dtype=jnp.float32)
    return pl.pallas_call(
        _router_kernel,
        grid=(B_, NS),
        in_specs=[
            pl.BlockSpec((1, BS, N_EMBED), lambda b, s: (b, s, 0)),
            pl.BlockSpec((B_, N_EMBED), lambda b, s: (0, 0)),
            pl.BlockSpec((E, N_EMBED), lambda b, s: (0, 0)),
            pl.BlockSpec((1, E), lambda b, s: (0, 0)),
            pl.BlockSpec((E, N_EMBED), lambda b, s: (0, 0)),
            pl.BlockSpec((E, 1), lambda b, s: (0, 0)),
            pl.BlockSpec((B_, E), lambda b, s: (0, 0)),
        ],
        out_specs=[
            pl.BlockSpec((B_, E), lambda b, s: (0, 0)),
            pl.BlockSpec((B_, TOP_K), lambda b, s: (0, 0)),
        ],
        out_shape=[
            jax.ShapeDtypeStruct((B_, E), jnp.float32),
            jax.ShapeDtypeStruct((B_, TOP_K), jnp.int32),
        ],
        scratch_shapes=[pltpu.VMEM((E, EP), jnp.float32)],
    )(mh_output, mh_output_avg, W_route, b_route[None, :], W_noise,
      b_noise[:, None], noise_sample)


# speedup vs baseline: 4.2419x; 1.0340x over previous
import jax
import jax.numpy as jnp
import numpy as np
from jax.experimental import pallas as pl
import jax.experimental.pallas.tpu as pltpu

N_EMBED = 1024
E = 8
EP = 128
TOP_K = 2
B_ = 4
S_ = 8192
BS = 2048
NS = S_ // BS

_NOISE = np.array([
    [-0.02830461598932743, 0.4671318531036377, 0.2957029640674591,
     0.15354591608047485, -0.12403281778097153, 0.21692314743995667,
     -1.440878987312317, 0.755859911441803],
    [0.5214096307754517, 0.9101703763008118, -0.3844965994358063,
     1.139823317527771, 1.4457862377166748, 1.080906629562378,
     -0.05629321187734604, 0.9095944762229919],
    [0.5573461651802063, 0.21905718743801117, -1.4485087394714355,
     0.7641875147819519, -0.24154697358608246, -1.179381012916565,
     -1.9389183521270752, 0.3562646210193634],
    [-0.24111966788768768, 1.2151274681091309, -1.3952220678329468,
     -0.5347688794136047, 0.27067556977272034, 1.5401241779327393,
     0.6935186386108398, -0.1038767620921135],
], dtype=np.float32)

def _router_kernel(x_ref, avg_ref, wr_ref, br_ref, wn_ref, bn_ref, ns_ref,
                   router_ref, idx_ref, acc_ref):
    b = pl.program_id(0)
    s = pl.program_id(1)

    @pl.when((b == 0) & (s == 0))
    def _init():
        acc_ref[...] = jnp.zeros_like(acc_ref)

    x = x_ref[0]
    yt = jax.lax.dot_general(wn_ref[...], x,
                             (((1,), (1,)), ((), ())),
                             preferred_element_type=jnp.float32)
    yt = yt + jnp.transpose(bn_ref[...])
    sp = jnp.maximum(yt, 0.0) + jnp.log1p(jnp.exp(-jnp.abs(yt)))
    part = jnp.sum(sp, axis=1, keepdims=True)
    lane = jax.lax.broadcasted_iota(jnp.int32, (E, EP), 1)
    acc_ref[...] += jnp.where(lane == b, part, 0.0)

    @pl.when((b == B_ - 1) & (s == NS - 1))
    def _epilogue():
        mean = jnp.transpose(acc_ref[...])[:B_, :E] * (1.0 / S_)
        logits = jax.lax.dot_general(avg_ref[...], wr_ref[...],
                                     (((1,), (1,)), ((), ())),
                                     preferred_element_type=jnp.float32)
        noisy = logits + br_ref[...] + ns_ref[...] * mean
        col = jax.lax.broadcasted_iota(jnp.int32, (B_, E), 1)
        neg = jnp.float32(-1e30)
        m1 = jnp.max(noisy, axis=1, keepdims=True)
        i1 = jnp.min(jnp.where(noisy == m1, col, E), axis=1, keepdims=True)
        rest = jnp.where(col == i1, neg, noisy)
        m2 = jnp.max(rest, axis=1, keepdims=True)
        i2 = jnp.min(jnp.where(rest == m2, col, E), axis=1, keepdims=True)
        d = jnp.exp(m2 - m1)
        p1 = 1.0 / (1.0 + d)
        p2 = d / (1.0 + d)
        router_ref[...] = jnp.where(col == i1, p1,
                                    jnp.where(col == i2, p2, 0.0))
        idx_ref[:, 0:1] = i1
        idx_ref[:, 1:2] = i2


def kernel(mh_output, mh_output_avg, W_route, b_route, W_noise, b_noise):
    return pl.pallas_call(
        _router_kernel,
        grid=(B_, NS),
        in_specs=[
            pl.BlockSpec((1, BS, N_EMBED), lambda b, s: (b, s, 0)),
            pl.BlockSpec((B_, N_EMBED), lambda b, s: (0, 0)),
            pl.BlockSpec((E, N_EMBED), lambda b, s: (0, 0)),
            pl.BlockSpec((1, E), lambda b, s: (0, 0)),
            pl.BlockSpec((E, N_EMBED), lambda b, s: (0, 0)),
            pl.BlockSpec((1, E), lambda b, s: (0, 0)),
            pl.BlockSpec((B_, E), lambda b, s: (0, 0)),
        ],
        out_specs=[
            pl.BlockSpec((B_, E), lambda b, s: (0, 0)),
            pl.BlockSpec((B_, TOP_K), lambda b, s: (0, 0)),
        ],
        out_shape=[
            jax.ShapeDtypeStruct((B_, E), jnp.float32),
            jax.ShapeDtypeStruct((B_, TOP_K), jnp.int32),
        ],
        scratch_shapes=[pltpu.VMEM((E, EP), jnp.float32)],
    )(mh_output, mh_output_avg, W_route, b_route[None, :], W_noise,
      b_noise[None, :], jnp.asarray(_NOISE))


# speedup vs baseline: 4.2639x; 1.0052x over previous
import jax
import jax.numpy as jnp
import numpy as np
from jax.experimental import pallas as pl
import jax.experimental.pallas.tpu as pltpu

N_EMBED = 1024
E = 8
EP = 128
TOP_K = 2
B_ = 4
S_ = 8192
BS = 2048
NS = S_ // BS

_NOISE = np.array([
    [-0.02830461598932743, 0.4671318531036377, 0.2957029640674591,
     0.15354591608047485, -0.12403281778097153, 0.21692314743995667,
     -1.440878987312317, 0.755859911441803],
    [0.5214096307754517, 0.9101703763008118, -0.3844965994358063,
     1.139823317527771, 1.4457862377166748, 1.080906629562378,
     -0.05629321187734604, 0.9095944762229919],
    [0.5573461651802063, 0.21905718743801117, -1.4485087394714355,
     0.7641875147819519, -0.24154697358608246, -1.179381012916565,
     -1.9389183521270752, 0.3562646210193634],
    [-0.24111966788768768, 1.2151274681091309, -1.3952220678329468,
     -0.5347688794136047, 0.27067556977272034, 1.5401241779327393,
     0.6935186386108398, -0.1038767620921135],
], dtype=np.float32)

def _router_kernel(x_ref, avg_ref, wr_ref, br_ref, wn_ref, bn_ref, ns_ref,
                   router_ref, idx_ref, acc_ref):
    b = pl.program_id(0)
    s = pl.program_id(1)

    @pl.when((b == 0) & (s == 0))
    def _init():
        acc_ref[...] = jnp.zeros_like(acc_ref)

    x = x_ref[0]
    yt = jax.lax.dot_general(wn_ref[...], x,
                             (((1,), (1,)), ((), ())),
                             preferred_element_type=jnp.float32)
    yt = yt + bn_ref[...].reshape(E, 1)
    sp = jnp.maximum(yt, 0.0) + jnp.log1p(jnp.exp(-jnp.abs(yt)))
    part = jnp.sum(sp, axis=1, keepdims=True)
    lane = jax.lax.broadcasted_iota(jnp.int32, (E, EP), 1)
    acc_ref[...] += jnp.where(lane == b, part, 0.0)

    @pl.when((b == B_ - 1) & (s == NS - 1))
    def _epilogue():
        mean = jnp.transpose(acc_ref[...])[:B_, :E] * (1.0 / S_)
        logits = jax.lax.dot_general(avg_ref[...], wr_ref[...],
                                     (((1,), (1,)), ((), ())),
                                     preferred_element_type=jnp.float32)
        noisy = (logits + br_ref[...].reshape(1, E)
                 + ns_ref[...] * mean)
        col = jax.lax.broadcasted_iota(jnp.int32, (B_, E), 1)
        neg = jnp.float32(-1e30)
        m1 = jnp.max(noisy, axis=1, keepdims=True)
        i1 = jnp.min(jnp.where(noisy == m1, col, E), axis=1, keepdims=True)
        rest = jnp.where(col == i1, neg, noisy)
        m2 = jnp.max(rest, axis=1, keepdims=True)
        i2 = jnp.min(jnp.where(rest == m2, col, E), axis=1, keepdims=True)
        d = jnp.exp(m2 - m1)
        p1 = 1.0 / (1.0 + d)
        p2 = d / (1.0 + d)
        router_ref[...] = jnp.where(col == i1, p1,
                                    jnp.where(col == i2, p2, 0.0))
        idx_ref[:, 0:1] = i1
        idx_ref[:, 1:2] = i2


def kernel(mh_output, mh_output_avg, W_route, b_route, W_noise, b_noise):
    return pl.pallas_call(
        _router_kernel,
        grid=(B_, NS),
        in_specs=[
            pl.BlockSpec((1, BS, N_EMBED), lambda b, s: (b, s, 0)),
            pl.BlockSpec((B_, N_EMBED), lambda b, s: (0, 0)),
            pl.BlockSpec((E, N_EMBED), lambda b, s: (0, 0)),
            pl.BlockSpec((E,), lambda b, s: (0,)),
            pl.BlockSpec((E, N_EMBED), lambda b, s: (0, 0)),
            pl.BlockSpec((E,), lambda b, s: (0,)),
            pl.BlockSpec((B_, E), lambda b, s: (0, 0)),
        ],
        out_specs=[
            pl.BlockSpec((B_, E), lambda b, s: (0, 0)),
            pl.BlockSpec((B_, TOP_K), lambda b, s: (0, 0)),
        ],
        out_shape=[
            jax.ShapeDtypeStruct((B_, E), jnp.float32),
            jax.ShapeDtypeStruct((B_, TOP_K), jnp.int32),
        ],
        scratch_shapes=[pltpu.VMEM((E, EP), jnp.float32)],
    )(mh_output, mh_output_avg, W_route, b_route, W_noise,
      b_noise, jnp.asarray(_NOISE))


# speedup vs baseline: 4.2900x; 1.0061x over previous
import jax
import jax.numpy as jnp
import numpy as np
from jax.experimental import pallas as pl
import jax.experimental.pallas.tpu as pltpu

N_EMBED = 1024
E = 8
EP = 128
TOP_K = 2
B_ = 4
S_ = 8192
BS = 2048
NS = S_ // BS

_NOISE = np.array([
    [-0.02830461598932743, 0.4671318531036377, 0.2957029640674591,
     0.15354591608047485, -0.12403281778097153, 0.21692314743995667,
     -1.440878987312317, 0.755859911441803],
    [0.5214096307754517, 0.9101703763008118, -0.3844965994358063,
     1.139823317527771, 1.4457862377166748, 1.080906629562378,
     -0.05629321187734604, 0.9095944762229919],
    [0.5573461651802063, 0.21905718743801117, -1.4485087394714355,
     0.7641875147819519, -0.24154697358608246, -1.179381012916565,
     -1.9389183521270752, 0.3562646210193634],
    [-0.24111966788768768, 1.2151274681091309, -1.3952220678329468,
     -0.5347688794136047, 0.27067556977272034, 1.5401241779327393,
     0.6935186386108398, -0.1038767620921135],
], dtype=np.float32)

def _router_kernel(x_ref, avg_ref, wr_ref, br_ref, wn_ref, bn_ref,
                   router_ref, idx_ref, acc_ref):
    b = pl.program_id(0)
    s = pl.program_id(1)

    @pl.when((b == 0) & (s == 0))
    def _init():
        acc_ref[...] = jnp.zeros_like(acc_ref)

    x = x_ref[0]
    yt = jax.lax.dot_general(wn_ref[...], x,
                             (((1,), (1,)), ((), ())),
                             preferred_element_type=jnp.float32)
    yt = yt + bn_ref[...].reshape(E, 1)
    sp = jnp.maximum(yt, 0.0) + jnp.log1p(jnp.exp(-jnp.abs(yt)))
    part = jnp.sum(sp, axis=1, keepdims=True)
    lane = jax.lax.broadcasted_iota(jnp.int32, (E, EP), 1)
    acc_ref[...] += jnp.where(lane == b, part, 0.0)

    @pl.when((b == B_ - 1) & (s == NS - 1))
    def _epilogue():
        mean = jnp.transpose(acc_ref[...])[:B_, :E] * (1.0 / S_)
        logits = jax.lax.dot_general(avg_ref[...], wr_ref[...],
                                     (((1,), (1,)), ((), ())),
                                     preferred_element_type=jnp.float32)
        col = jax.lax.broadcasted_iota(jnp.int32, (B_, E), 1)
        row = jax.lax.broadcasted_iota(jnp.int32, (B_, E), 0)
        ns = jnp.zeros((B_, E), jnp.float32)
        for bb in range(B_):
            for ee in range(E):
                ns = jnp.where((row == bb) & (col == ee),
                               float(_NOISE[bb, ee]), ns)
        noisy = (logits + br_ref[...].reshape(1, E)
                 + ns * mean)
        neg = jnp.float32(-1e30)
        m1 = jnp.max(noisy, axis=1, keepdims=True)
        i1 = jnp.min(jnp.where(noisy == m1, col, E), axis=1, keepdims=True)
        rest = jnp.where(col == i1, neg, noisy)
        m2 = jnp.max(rest, axis=1, keepdims=True)
        i2 = jnp.min(jnp.where(rest == m2, col, E), axis=1, keepdims=True)
        d = jnp.exp(m2 - m1)
        p1 = 1.0 / (1.0 + d)
        p2 = d / (1.0 + d)
        router_ref[...] = jnp.where(col == i1, p1,
                                    jnp.where(col == i2, p2, 0.0))
        idx_ref[:, 0:1] = i1
        idx_ref[:, 1:2] = i2


def kernel(mh_output, mh_output_avg, W_route, b_route, W_noise, b_noise):
    return pl.pallas_call(
        _router_kernel,
        grid=(B_, NS),
        in_specs=[
            pl.BlockSpec((1, BS, N_EMBED), lambda b, s: (b, s, 0)),
            pl.BlockSpec((B_, N_EMBED), lambda b, s: (0, 0)),
            pl.BlockSpec((E, N_EMBED), lambda b, s: (0, 0)),
            pl.BlockSpec((E,), lambda b, s: (0,)),
            pl.BlockSpec((E, N_EMBED), lambda b, s: (0, 0)),
            pl.BlockSpec((E,), lambda b, s: (0,)),
        ],
        out_specs=[
            pl.BlockSpec((B_, E), lambda b, s: (0, 0)),
            pl.BlockSpec((B_, TOP_K), lambda b, s: (0, 0)),
        ],
        out_shape=[
            jax.ShapeDtypeStruct((B_, E), jnp.float32),
            jax.ShapeDtypeStruct((B_, TOP_K), jnp.int32),
        ],
        scratch_shapes=[pltpu.VMEM((E, EP), jnp.float32)],
    )(mh_output, mh_output_avg, W_route, b_route, W_noise, b_noise)


# speedup vs baseline: 4.3838x; 1.0219x over previous
import jax
import jax.numpy as jnp
import numpy as np
from jax.experimental import pallas as pl
import jax.experimental.pallas.tpu as pltpu

N_EMBED = 1024
E = 8
EP = 128
TOP_K = 2
B_ = 4
S_ = 8192
BS = 1024
NS = S_ // (2 * BS)

_NOISE = np.array([
    [-0.02830461598932743, 0.4671318531036377, 0.2957029640674591,
     0.15354591608047485, -0.12403281778097153, 0.21692314743995667,
     -1.440878987312317, 0.755859911441803],
    [0.5214096307754517, 0.9101703763008118, -0.3844965994358063,
     1.139823317527771, 1.4457862377166748, 1.080906629562378,
     -0.05629321187734604, 0.9095944762229919],
    [0.5573461651802063, 0.21905718743801117, -1.4485087394714355,
     0.7641875147819519, -0.24154697358608246, -1.179381012916565,
     -1.9389183521270752, 0.3562646210193634],
    [-0.24111966788768768, 1.2151274681091309, -1.3952220678329468,
     -0.5347688794136047, 0.27067556977272034, 1.5401241779327393,
     0.6935186386108398, -0.1038767620921135],
], dtype=np.float32)

def _router_kernel(x_ref, x2_ref, avg_ref, wr_ref, br_ref, wn_ref, bn_ref,
                   router_ref, idx_ref, acc_ref):
    b = pl.program_id(0)
    s = pl.program_id(1)

    @pl.when((b == 0) & (s == 0))
    def _init():
        acc_ref[...] = jnp.zeros_like(acc_ref)

    dnum = (((1,), (1,)), ((), ()))
    bn = bn_ref[...].reshape(E, 1)
    ya = jax.lax.dot_general(wn_ref[...], x_ref[0], dnum,
                             preferred_element_type=jnp.float32) + bn
    yb = jax.lax.dot_general(wn_ref[...], x2_ref[0], dnum,
                             preferred_element_type=jnp.float32) + bn
    spa = jnp.maximum(ya, 0.0) + jnp.log1p(jnp.exp(-jnp.abs(ya)))
    spb = jnp.maximum(yb, 0.0) + jnp.log1p(jnp.exp(-jnp.abs(yb)))
    part = (jnp.sum(spa, axis=1, keepdims=True)
            + jnp.sum(spb, axis=1, keepdims=True))
    lane = jax.lax.broadcasted_iota(jnp.int32, (E, EP), 1)
    acc_ref[...] += jnp.where(lane == b, part, 0.0)

    @pl.when((b == B_ - 1) & (s == NS - 1))
    def _epilogue():
        mean = jnp.transpose(acc_ref[...])[:B_, :E] * (1.0 / S_)
        logits = jax.lax.dot_general(avg_ref[...], wr_ref[...],
                                     (((1,), (1,)), ((), ())),
                                     preferred_element_type=jnp.float32)
        col = jax.lax.broadcasted_iota(jnp.int32, (B_, E), 1)
        row = jax.lax.broadcasted_iota(jnp.int32, (B_, E), 0)
        ns = jnp.zeros((B_, E), jnp.float32)
        for bb in range(B_):
            for ee in range(E):
                ns = jnp.where((row == bb) & (col == ee),
                               float(_NOISE[bb, ee]), ns)
        noisy = (logits + br_ref[...].reshape(1, E)
                 + ns * mean)
        neg = jnp.float32(-1e30)
        m1 = jnp.max(noisy, axis=1, keepdims=True)
        i1 = jnp.min(jnp.where(noisy == m1, col, E), axis=1, keepdims=True)
        rest = jnp.where(col == i1, neg, noisy)
        m2 = jnp.max(rest, axis=1, keepdims=True)
        i2 = jnp.min(jnp.where(rest == m2, col, E), axis=1, keepdims=True)
        d = jnp.exp(m2 - m1)
        p1 = 1.0 / (1.0 + d)
        p2 = d / (1.0 + d)
        router_ref[...] = jnp.where(col == i1, p1,
                                    jnp.where(col == i2, p2, 0.0))
        idx_ref[:, 0:1] = i1
        idx_ref[:, 1:2] = i2


def kernel(mh_output, mh_output_avg, W_route, b_route, W_noise, b_noise):
    return pl.pallas_call(
        _router_kernel,
        grid=(B_, NS),
        in_specs=[
            pl.BlockSpec((1, BS, N_EMBED), lambda b, s: (b, 2 * s, 0)),
            pl.BlockSpec((1, BS, N_EMBED), lambda b, s: (b, 2 * s + 1, 0)),
            pl.BlockSpec((B_, N_EMBED), lambda b, s: (0, 0)),
            pl.BlockSpec((E, N_EMBED), lambda b, s: (0, 0)),
            pl.BlockSpec((E,), lambda b, s: (0,)),
            pl.BlockSpec((E, N_EMBED), lambda b, s: (0, 0)),
            pl.BlockSpec((E,), lambda b, s: (0,)),
        ],
        out_specs=[
            pl.BlockSpec((B_, E), lambda b, s: (0, 0)),
            pl.BlockSpec((B_, TOP_K), lambda b, s: (0, 0)),
        ],
        out_shape=[
            jax.ShapeDtypeStruct((B_, E), jnp.float32),
            jax.ShapeDtypeStruct((B_, TOP_K), jnp.int32),
        ],
        scratch_shapes=[pltpu.VMEM((E, EP), jnp.float32)],
    )(mh_output, mh_output, mh_output_avg, W_route, b_route, W_noise,
      b_noise)
